# baseline (speedup 1.0000x reference)
import functools
import math

import jax
import jax.numpy as jnp
from jax import lax
from jax.experimental import pallas as pl
from jax.experimental.pallas import tpu as pltpu
from jax.experimental.pallas import tpu_sc as plsc

D = 32
NR = 1000000
BB, LL = 4096, 200
NC, NS = 2, 16
NW = NC * NS
BT = BB // 128
CL = 4
CHUNK = CL * 128
NCH = LL // CL
PAD = 129
SCALE = math.sqrt(D)

NT = (NR + 127) // 128
NTF = NT - 1
JFULL = NTF // NW

_mesh = plsc.VectorSubcoreMesh(
    core_axis_name="c", subcore_axis_name="s", num_cores=NC, num_subcores=NS
)


@functools.partial(
    pl.kernel,
    out_type=jax.ShapeDtypeStruct((NR * D // 128, 128), jnp.float32),
    mesh=_mesh,
    scratch_types=[
        pltpu.VMEM((D, 128), jnp.float32),
        pltpu.VMEM((D, 128), jnp.float32),
        pltpu.VMEM((D, 128), jnp.float32),
        pltpu.VMEM((D, 128), jnp.float32),
        pltpu.SemaphoreType.DMA,
        pltpu.SemaphoreType.DMA,
        pltpu.SemaphoreType.DMA,
        pltpu.SemaphoreType.DMA,
    ],
    compiler_params=pltpu.CompilerParams(
        use_tc_tiling_on_sc=True, needs_layout_passes=False
    ),
)
def _detile_table(tv_hbm, rem_hbm, out_hbm, b0, b1, s0, s1, i0, i1, o0, o1):
    wid = lax.axis_index("s") * NC + lax.axis_index("c")
    bufs = (b0, b1)
    scr = (s0, s1)
    isem = (i0, i1)
    osem = (o0, o1)
    iota = lax.iota(jnp.int32, 16)
    ev = (iota, iota + 16)

    def tcol(j):
        return wid + NW * j

    def start_in(j, k):
        c0 = pl.multiple_of(128 * tcol(j), 128)
        pltpu.async_copy(tv_hbm.at[:, pl.ds(c0, 128)], bufs[k], isem[k])

    def wait_in(j, k):
        c0 = pl.multiple_of(128 * tcol(j), 128)
        pltpu.make_async_copy(
            tv_hbm.at[:, pl.ds(c0, 128)], bufs[k], isem[k]
        ).wait()

    def transpose(k):
        @plsc.parallel_loop(0, 16, 1, unroll=4)
        def _(d):
            md = (iota + d) & 15
            mr = md >> 2
            cd0 = (md & 3) * 32 + iota
            cd1 = cd0 + 16
            for cbl in range(8):
                colv = md + 16 * cbl
                rdst = mr + 4 * cbl
                v0 = plsc.load_gather(bufs[k], [ev[0], colv])
                plsc.store_scatter(scr[k], [rdst, cd0], v0)
                v1 = plsc.load_gather(bufs[k], [ev[1], colv])
                plsc.store_scatter(scr[k], [rdst, cd1], v1)

    def start_out(j, k):
        r0 = pl.multiple_of(32 * tcol(j), 32)
        pltpu.async_copy(scr[k], out_hbm.at[pl.ds(r0, 32)], osem[k])

    def wait_out(j, k):
        r0 = pl.multiple_of(32 * tcol(j), 32)
        pltpu.make_async_copy(
            scr[k], out_hbm.at[pl.ds(r0, 32)], osem[k]
        ).wait()

    start_in(0, 0)
    start_in(1, 1)

    for j in range(2):
        wait_in(j, j)
        transpose(j)
        start_in(j + 2, j)
        start_out(j, j)

    def body(j2, _):
        for k in range(2):
            j = 2 * j2 + k
            wait_in(j, k)
            wait_out(j - 2, k)
            transpose(k)
            start_in(j + 2, k)
            start_out(j, k)
        return 0

    lax.fori_loop(1, JFULL // 2 - 1, body, 0)

    for j in (JFULL - 2, JFULL - 1):
        k = j % 2
        wait_in(j, k)
        wait_out(j - 2, k)
        transpose(k)
        start_out(j, k)

    jt = JFULL

    @pl.when(wid < 4)
    def _():
        start_in(jt, 0)
        wait_in(jt, 0)
        wait_out(jt - 2, 0)
        transpose(0)
        start_out(jt, 0)
        wait_out(jt, 0)

    @pl.when(wid == 4)
    def _():
        wait_out(jt - 2, 0)
        pltpu.async_copy(rem_hbm, out_hbm.at[pl.ds(NTF * 32, 16)], osem[0])
        pltpu.make_async_copy(
            rem_hbm, out_hbm.at[pl.ds(NTF * 32, 16)], osem[0]
        ).wait()

    @pl.when(wid >= 5)
    def _():
        wait_out(jt - 2, 0)

    wait_out(jt - 1, 1)


@functools.partial(
    pl.kernel,
    out_type=jax.ShapeDtypeStruct((LL, 4, BT, 8, 128), jnp.float32),
    mesh=_mesh,
    scratch_types=[
        pltpu.VMEM((128, LL), jnp.int32),
        pltpu.VMEM((CHUNK,), jnp.int32),
        pltpu.VMEM((CHUNK,), jnp.int32),
        pltpu.VMEM((CHUNK,), jnp.int32),
        pltpu.VMEM((CHUNK, D), jnp.float32),
        pltpu.VMEM((CHUNK, D), jnp.float32),
        pltpu.VMEM((CHUNK, D), jnp.float32),
        pltpu.VMEM((CL, 4, 8, PAD), jnp.float32),
        pltpu.VMEM((CL, 4, 8, PAD), jnp.float32),
        pltpu.SemaphoreType.DMA,
        pltpu.SemaphoreType.DMA,
        pltpu.SemaphoreType.DMA,
        pltpu.SemaphoreType.DMA,
        pltpu.SemaphoreType.DMA,
    ],
    compiler_params=pltpu.CompilerParams(
        use_tc_tiling_on_sc=False, needs_layout_passes=False
    ),
)
def _embed_lookup(x_hbm, table_hbm, out_hbm,
                  xblk, idx0, idx1, idx2, rows0, rows1, rows2, t0, t1,
                  g0, g1, g2, o0, o1):
    wid = lax.axis_index("s") * NC + lax.axis_index("c")
    idxs = (idx0, idx1, idx2)
    rows = (rows0, rows1, rows2)
    gsem = (g0, g1, g2)
    ts = (t0, t1)
    osem = (o0, o1)

    iota = lax.iota(jnp.int32, 16)
    e0v = iota & 7
    gv0 = iota >> 3
    gv1 = gv0 + 2

    def build_idx(c, k):
        for lc in range(CL):
            l = c * CL + lc
            colv = jnp.broadcast_to(l, (16,)).astype(jnp.int32)
            for bs in range(8):
                rv = plsc.load_gather(xblk, [bs * 16 + iota, colv])
                idxs[k][pl.ds(lc * 128 + bs * 16, 16)] = rv

    def start_gather(k):
        pltpu.async_copy(table_hbm.at[idxs[k]], rows[k], gsem[k])

    def wait_gather(k):
        pltpu.make_async_copy(table_hbm.at[idxs[k]], rows[k], gsem[k]).wait()

    def transpose(k, tk):
        @plsc.parallel_loop(0, CHUNK, 1, unroll=4)
        def _(i):
            lc = i >> 7
            bb = i & 127
            lcv = jnp.broadcast_to(lc, (16,))
            bv = jnp.broadcast_to(bb, (16,))
            v0 = rows[k][i, pl.ds(0, 16)] * SCALE
            v1 = rows[k][i, pl.ds(16, 16)] * SCALE
            plsc.store_scatter(ts[tk], [lcv, gv0, e0v, bv], v0)
            plsc.store_scatter(ts[tk], [lcv, gv1, e0v, bv], v1)

    def start_out(c, tk):
        for lc in range(CL):
            l = c * CL + lc
            for g in range(4):
                pltpu.async_copy(
                    ts[tk].at[lc, g, :, pl.ds(0, 128)],
                    out_hbm.at[l, g, wid],
                    osem[tk],
                )

    def wait_out(c, tk):
        for lc in range(CL):
            l = c * CL + lc
            for g in range(4):
                pltpu.make_async_copy(
                    ts[tk].at[lc, g, :, pl.ds(0, 128)],
                    out_hbm.at[l, g, wid],
                    osem[tk],
                ).wait()

    def chunk_body(c, rb, with_build=True, with_outwait=True):
        nb = (rb + 2) % 3
        tb = c % 2 if isinstance(c, int) else None
        wait_gather(rb)
        if with_build:
            build_idx(c + 2, nb)
            start_gather(nb)
        if with_outwait:
            wait_out(c - 2, tb)
        transpose(rb, tb)
        start_out(c, tb)

    pltpu.sync_copy(x_hbm.at[pl.ds(wid * 128, 128)], xblk)

    build_idx(0, 0)
    start_gather(0)
    build_idx(1, 1)
    start_gather(1)

    chunk_body(0, 0, with_outwait=False)
    chunk_body(1, 1, with_outwait=False)

    def group(s, _):
        for k in range(6):
            c = 2 + s * 6 + k
            rb = (2 + k) % 3
            nb = (rb + 2) % 3
            tb = k % 2
            wait_gather(rb)
            build_idx(c + 2, nb)
            start_gather(nb)
            wait_out(c - 2, tb)
            transpose(rb, tb)
            start_out(c, tb)
        return 0

    lax.fori_loop(0, 7, group, 0)

    chunk_body(44, 2)
    chunk_body(45, 0)
    chunk_body(46, 1)
    chunk_body(47, 2)

    chunk_body(48, 0, with_build=False)
    chunk_body(49, 1, with_build=False)

    wait_out(48, 0)
    wait_out(49, 1)


def kernel(x, table):
    tv = table.T
    rem = table[NTF * 128:].reshape(16, 128)
    lin = _detile_table(tv, rem)
    a = _embed_lookup(x, lin.reshape(NR, D))
    return a.transpose(2, 4, 0, 1, 3).reshape(BB, LL, D)


# speedup vs baseline: 1.1283x; 1.1283x over previous
import functools
import math

import jax
import jax.numpy as jnp
from jax import lax
from jax.experimental import pallas as pl
from jax.experimental.pallas import tpu as pltpu
from jax.experimental.pallas import tpu_sc as plsc

D = 32
NR = 1000000
BB, LL = 4096, 200
NC, NS = 2, 16
NW = NC * NS
BT = BB // 128
CL = 4
CHUNK = CL * 128
NCH = LL // CL
PAD = 129
SCALE = math.sqrt(D)

NT = (NR + 127) // 128
NTF = NT - 1
JFULL = NTF // NW

_mesh = plsc.VectorSubcoreMesh(
    core_axis_name="c", subcore_axis_name="s", num_cores=NC, num_subcores=NS
)


@functools.partial(
    pl.kernel,
    out_type=jax.ShapeDtypeStruct((NR * D // 128, 128), jnp.float32),
    mesh=_mesh,
    scratch_types=[
        pltpu.VMEM((D, 128), jnp.float32),
        pltpu.VMEM((D, 128), jnp.float32),
        pltpu.VMEM((D, 128), jnp.float32),
        pltpu.VMEM((D, 128), jnp.float32),
        pltpu.SemaphoreType.DMA,
        pltpu.SemaphoreType.DMA,
        pltpu.SemaphoreType.DMA,
        pltpu.SemaphoreType.DMA,
    ],
    compiler_params=pltpu.CompilerParams(
        use_tc_tiling_on_sc=True, needs_layout_passes=False
    ),
)
def _detile_table(tv_hbm, rem_hbm, out_hbm, b0, b1, s0, s1, i0, i1, o0, o1):
    wid = lax.axis_index("s") * NC + lax.axis_index("c")
    bufs = (b0, b1)
    scr = (s0, s1)
    isem = (i0, i1)
    osem = (o0, o1)
    iota = lax.iota(jnp.int32, 16)
    ev = (iota, iota + 16)

    def tcol(j):
        return wid + NW * j

    def start_in(j, k):
        c0 = pl.multiple_of(128 * tcol(j), 128)
        pltpu.async_copy(tv_hbm.at[:, pl.ds(c0, 128)], bufs[k], isem[k])

    def wait_in(j, k):
        c0 = pl.multiple_of(128 * tcol(j), 128)
        pltpu.make_async_copy(
            tv_hbm.at[:, pl.ds(c0, 128)], bufs[k], isem[k]
        ).wait()

    def transpose(k):
        @plsc.parallel_loop(0, 16, 1, unroll=2)
        def _(d):
            md = (iota + d) & 15
            mr = md >> 2
            cd0 = (md & 3) * 32 + iota
            cd1 = cd0 + 16
            for cbl in range(8):
                colv = md + 16 * cbl
                rdst = mr + 4 * cbl
                v0 = plsc.load_gather(bufs[k], [ev[0], colv])
                plsc.store_scatter(scr[k], [rdst, cd0], v0)
                v1 = plsc.load_gather(bufs[k], [ev[1], colv])
                plsc.store_scatter(scr[k], [rdst, cd1], v1)

    def start_out(j, k):
        r0 = pl.multiple_of(32 * tcol(j), 32)
        pltpu.async_copy(scr[k], out_hbm.at[pl.ds(r0, 32)], osem[k])

    def wait_out(j, k):
        r0 = pl.multiple_of(32 * tcol(j), 32)
        pltpu.make_async_copy(
            scr[k], out_hbm.at[pl.ds(r0, 32)], osem[k]
        ).wait()

    start_in(0, 0)
    start_in(1, 1)

    for j in range(2):
        wait_in(j, j)
        transpose(j)
        start_in(j + 2, j)
        start_out(j, j)

    def body(j2, _):
        for k in range(2):
            j = 2 * j2 + k
            wait_in(j, k)
            wait_out(j - 2, k)
            transpose(k)
            start_in(j + 2, k)
            start_out(j, k)
        return 0

    lax.fori_loop(1, JFULL // 2 - 1, body, 0)

    for j in (JFULL - 2, JFULL - 1):
        k = j % 2
        wait_in(j, k)
        wait_out(j - 2, k)
        transpose(k)
        start_out(j, k)

    jt = JFULL

    @pl.when(wid < 4)
    def _():
        start_in(jt, 0)
        wait_in(jt, 0)
        wait_out(jt - 2, 0)
        transpose(0)
        start_out(jt, 0)
        wait_out(jt, 0)

    @pl.when(wid == 4)
    def _():
        wait_out(jt - 2, 0)
        pltpu.async_copy(rem_hbm, out_hbm.at[pl.ds(NTF * 32, 16)], osem[0])
        pltpu.make_async_copy(
            rem_hbm, out_hbm.at[pl.ds(NTF * 32, 16)], osem[0]
        ).wait()

    @pl.when(wid >= 5)
    def _():
        wait_out(jt - 2, 0)

    wait_out(jt - 1, 1)


@functools.partial(
    pl.kernel,
    out_type=jax.ShapeDtypeStruct((LL, 4, BT, 8, 128), jnp.float32),
    mesh=_mesh,
    scratch_types=[
        pltpu.VMEM((128, LL), jnp.int32),
        pltpu.VMEM((CHUNK,), jnp.int32),
        pltpu.VMEM((CHUNK,), jnp.int32),
        pltpu.VMEM((CHUNK,), jnp.int32),
        pltpu.VMEM((CHUNK, D), jnp.float32),
        pltpu.VMEM((CHUNK, D), jnp.float32),
        pltpu.VMEM((CHUNK, D), jnp.float32),
        pltpu.VMEM((CL, 4, 8, PAD), jnp.float32),
        pltpu.VMEM((CL, 4, 8, PAD), jnp.float32),
        pltpu.SemaphoreType.DMA,
        pltpu.SemaphoreType.DMA,
        pltpu.SemaphoreType.DMA,
        pltpu.SemaphoreType.DMA,
        pltpu.SemaphoreType.DMA,
    ],
    compiler_params=pltpu.CompilerParams(
        use_tc_tiling_on_sc=False, needs_layout_passes=False
    ),
)
def _embed_lookup(x_hbm, table_hbm, out_hbm,
                  xblk, idx0, idx1, idx2, rows0, rows1, rows2, t0, t1,
                  g0, g1, g2, o0, o1):
    wid = lax.axis_index("s") * NC + lax.axis_index("c")
    idxs = (idx0, idx1, idx2)
    rows = (rows0, rows1, rows2)
    gsem = (g0, g1, g2)
    ts = (t0, t1)
    osem = (o0, o1)

    iota = lax.iota(jnp.int32, 16)
    e0v = iota & 7
    gv0 = iota >> 3
    gv1 = gv0 + 2

    def build_idx(c, k):
        for lc in range(CL):
            l = c * CL + lc
            colv = jnp.broadcast_to(l, (16,)).astype(jnp.int32)
            for bs in range(8):
                rv = plsc.load_gather(xblk, [bs * 16 + iota, colv])
                idxs[k][pl.ds(lc * 128 + bs * 16, 16)] = rv

    def start_gather(k):
        pltpu.async_copy(table_hbm.at[idxs[k]], rows[k], gsem[k])

    def wait_gather(k):
        pltpu.make_async_copy(table_hbm.at[idxs[k]], rows[k], gsem[k]).wait()

    def transpose(k, tk):
        @plsc.parallel_loop(0, CHUNK, 1, unroll=4)
        def _(i):
            lc = i >> 7
            bb = i & 127
            lcv = jnp.broadcast_to(lc, (16,))
            bv = jnp.broadcast_to(bb, (16,))
            v0 = rows[k][i, pl.ds(0, 16)] * SCALE
            v1 = rows[k][i, pl.ds(16, 16)] * SCALE
            plsc.store_scatter(ts[tk], [lcv, gv0, e0v, bv], v0)
            plsc.store_scatter(ts[tk], [lcv, gv1, e0v, bv], v1)

    def start_out(c, tk):
        for lc in range(CL):
            l = c * CL + lc
            for g in range(4):
                pltpu.async_copy(
                    ts[tk].at[lc, g, :, pl.ds(0, 128)],
                    out_hbm.at[l, g, wid],
                    osem[tk],
                )

    def wait_out(c, tk):
        for lc in range(CL):
            l = c * CL + lc
            for g in range(4):
                pltpu.make_async_copy(
                    ts[tk].at[lc, g, :, pl.ds(0, 128)],
                    out_hbm.at[l, g, wid],
                    osem[tk],
                ).wait()

    def chunk_body(c, rb, with_build=True, with_outwait=True):
        nb = (rb + 2) % 3
        tb = c % 2 if isinstance(c, int) else None
        wait_gather(rb)
        if with_build:
            build_idx(c + 2, nb)
            start_gather(nb)
        if with_outwait:
            wait_out(c - 2, tb)
        transpose(rb, tb)
        start_out(c, tb)

    pltpu.sync_copy(x_hbm.at[pl.ds(wid * 128, 128)], xblk)

    build_idx(0, 0)
    start_gather(0)
    build_idx(1, 1)
    start_gather(1)

    chunk_body(0, 0, with_outwait=False)
    chunk_body(1, 1, with_outwait=False)

    def group(s, _):
        for k in range(6):
            c = 2 + s * 6 + k
            rb = (2 + k) % 3
            nb = (rb + 2) % 3
            tb = k % 2
            wait_gather(rb)
            build_idx(c + 2, nb)
            start_gather(nb)
            wait_out(c - 2, tb)
            transpose(rb, tb)
            start_out(c, tb)
        return 0

    lax.fori_loop(0, 7, group, 0)

    chunk_body(44, 2)
    chunk_body(45, 0)
    chunk_body(46, 1)
    chunk_body(47, 2)

    chunk_body(48, 0, with_build=False)
    chunk_body(49, 1, with_build=False)

    wait_out(48, 0)
    wait_out(49, 1)


def kernel(x, table):
    tv = table.T
    rem = table[NTF * 128:].reshape(16, 128)
    lin = _detile_table(tv, rem)
    a = _embed_lookup(x, lin.reshape(NR, D))
    return a.transpose(2, 4, 0, 1, 3).reshape(BB, LL, D)
